# Initial kernel scaffold; baseline (speedup 1.0000x reference)
import jax
import jax.numpy as jnp
from jax.experimental import pallas as pl


def kernel(x, edge_index, batch, ptr, params):
    raise NotImplementedError("write your pallas kernel here")


# speedup vs baseline: 1.0043x; 1.0043x over previous
import jax
import jax.numpy as jnp
from jax.experimental import pallas as pl

N_LAYERS = 4


def _ln(x, g, b):
    m = jnp.mean(x, axis=-1, keepdims=True)
    v = jnp.mean((x - m) ** 2, axis=-1, keepdims=True)
    return (x - m) / jnp.sqrt(v + 1e-5) * g + b


def _sage(x, src, dst, Wl, bl, Wr, n):
    xj = jnp.take(x, src, axis=0)
    ones = jnp.ones((src.shape[0],), x.dtype)
    cnt = jax.ops.segment_sum(ones, dst, num_segments=n)
    s = jax.ops.segment_sum(xj, dst, num_segments=n)
    mean = s / jnp.maximum(cnt, 1.0)[:, None]
    mx = jax.ops.segment_max(xj, dst, num_segments=n)
    mx = jnp.where(jnp.isfinite(mx), mx, 0.0)
    agg = jnp.concatenate([mx, mean], axis=-1)
    return agg @ Wl.T + bl + x @ Wr.T


def _readout_body(o_ref, w1_ref, b1_ref, g_ref, be_ref, w2_ref, b2_ref, out_ref):
    o = o_ref[...]
    h = o @ w1_ref[...].T + b1_ref[...]
    h = h * jax.nn.sigmoid(h)
    m = jnp.mean(h, axis=-1, keepdims=True)
    v = jnp.mean((h - m) ** 2, axis=-1, keepdims=True)
    h = (h - m) / jnp.sqrt(v + 1e-5) * g_ref[...] + be_ref[...]
    out_ref[...] = h @ w2_ref[...].T + b2_ref[...]


def kernel(x, edge_index, batch, ptr, params):
    src = edge_index[0]
    dst = edge_index[1]
    n = x.shape[0]
    b = ptr.shape[0] - 1
    x0 = x
    x_res = x @ params['res_W'].T + params['res_b']
    h = _sage(x, src, dst, params['init_Wl'], params['init_bl'], params['init_Wr'], n)
    h = _ln(h, params['ln_g'][0], params['ln_b'][0])
    h = jax.nn.silu(h + x_res)
    for i in range(N_LAYERS - 1):
        r = h
        h = _sage(h, src, dst, params['conv_Wl'][i], params['conv_bl'][i], params['conv_Wr'][i], n)
        h = _ln(h, params['ln_g'][i + 1], params['ln_b'][i + 1])
        h = jax.nn.silu(h + r)
    m = jax.nn.silu(h @ params['mlp_W1'].T + params['mlp_b1'])
    m = _ln(m, params['mlp_g'], params['mlp_be'])
    m = m @ params['mlp_W2'].T + params['mlp_b2']
    cnt = jax.ops.segment_sum(jnp.ones((n,), x.dtype), batch, num_segments=b)
    psum = jax.ops.segment_sum(m, batch, num_segments=b)
    pmean = psum / jnp.maximum(cnt, 1.0)[:, None]
    pmax = jax.ops.segment_max(m, batch, num_segments=b)
    pmax = jnp.where(jnp.isfinite(pmax), pmax, 0.0)
    root = jnp.take(x0, ptr[:-1], axis=0)
    o = jnp.concatenate([pmean, pmax, psum, root], axis=-1)
    out = pl.pallas_call(
        _readout_body,
        out_shape=jax.ShapeDtypeStruct((b, params['ro_W2'].shape[0]), jnp.float32),
    )(o, params['ro_W1'], params['ro_b1'], params['ro_g'], params['ro_be'],
      params['ro_W2'], params['ro_b2'])
    return out


# speedup vs baseline: 1.0465x; 1.0420x over previous
import jax
import jax.numpy as jnp
from jax.experimental import pallas as pl

N = 100000
G = 100
HID = 32
N_LAYERS = 4


def _ln(x, g, b):
    mu = jnp.mean(x, axis=-1, keepdims=True)
    v = jnp.mean((x - mu) ** 2, axis=-1, keepdims=True)
    return (x - mu) / jnp.sqrt(v + 1e-5) * g + b


def _dense_body(h_ref, sm_ref, mx_ref, s1_ref, wl_ref, bl_ref, wr_ref,
                rw_ref, rb_ref, g_ref, b_ref, out_ref):
    h = h_ref[...]
    sm = sm_ref[...]
    mx = mx_ref[...]
    cnt = s1_ref[...][:, 4:5]
    mean = sm / jnp.maximum(cnt, 1.0)
    mx = jnp.where(jnp.isfinite(mx), mx, 0.0)
    agg = jnp.concatenate([mx, mean], axis=-1)
    z = (jnp.dot(agg, wl_ref[...].T, preferred_element_type=jnp.float32)
         + bl_ref[...]
         + jnp.dot(h, wr_ref[...].T, preferred_element_type=jnp.float32))
    res = (jnp.dot(h, rw_ref[...].T, preferred_element_type=jnp.float32)
           + rb_ref[...])
    z = _ln(z, g_ref[...], b_ref[...]) + res
    out_ref[...] = z * jax.nn.sigmoid(z)


def _dense(h, sm, mx, s1, wl, bl, wr, rw, rb, g, b):
    blk = 1000
    grid = N // blk
    full = lambda a: pl.BlockSpec(a.shape, lambda i: (0,) * a.ndim)
    row = pl.BlockSpec((blk, HID), lambda i: (i, 0))
    return pl.pallas_call(
        _dense_body,
        grid=(grid,),
        in_specs=[row, row, row, row, full(wl), full(bl), full(wr),
                  full(rw), full(rb), full(g), full(b)],
        out_specs=row,
        out_shape=jax.ShapeDtypeStruct((N, HID), jnp.float32),
    )(h, sm, mx, s1, wl, bl, wr, rw, rb, g, b)


def _mlp_body(h_ref, w1_ref, b1_ref, g_ref, be_ref, w2_ref, b2_ref, out_ref):
    h = h_ref[...]
    z = jnp.dot(h, w1_ref[...].T, preferred_element_type=jnp.float32) + b1_ref[...]
    z = z * jax.nn.sigmoid(z)
    z = _ln(z, g_ref[...], be_ref[...])
    out_ref[...] = (jnp.dot(z, w2_ref[...].T, preferred_element_type=jnp.float32)
                    + b2_ref[...])


def _mlp(h, w1, b1, g, be, w2, b2):
    blk = 1000
    grid = N // blk
    full = lambda a: pl.BlockSpec(a.shape, lambda i: (0,) * a.ndim)
    row = pl.BlockSpec((blk, HID), lambda i: (i, 0))
    return pl.pallas_call(
        _mlp_body,
        grid=(grid,),
        in_specs=[row, full(w1), full(b1), full(g), full(be), full(w2),
                  full(b2)],
        out_specs=row,
        out_shape=jax.ShapeDtypeStruct((N, HID), jnp.float32),
    )(h, w1, b1, g, be, w2, b2)


def _readout_body(ps_ref, pm_ref, rt_ref, ptr_ref, w1_ref, b1_ref, g_ref,
                  be_ref, w2_ref, b2_ref, out_ref):
    ps = ps_ref[...]
    pm = pm_ref[...]
    rt = rt_ref[...]
    ptrf = ptr_ref[...]
    cnt = jnp.concatenate([ptrf[1:], ptrf[:1]]) - ptrf
    pmean = ps / jnp.maximum(cnt, 1.0)[:, None]
    pm = jnp.where(jnp.isfinite(pm), pm, 0.0)
    o = jnp.concatenate([pmean, pm, ps, rt], axis=-1)
    z = jnp.dot(o, w1_ref[...].T, preferred_element_type=jnp.float32) + b1_ref[...]
    z = z * jax.nn.sigmoid(z)
    z = _ln(z, g_ref[...], be_ref[...])
    out_ref[...] = (jnp.dot(z, w2_ref[...].T, preferred_element_type=jnp.float32)
                    + b2_ref[...])


def _readout(ps, pm, rt, ptrf, w1, b1, g, be, w2, b2):
    return pl.pallas_call(
        _readout_body,
        out_shape=jax.ShapeDtypeStruct((128, w2.shape[0]), jnp.float32),
    )(ps, pm, rt, ptrf, w1, b1, g, be, w2, b2)


def kernel(x, edge_index, batch, ptr, params):
    p = params
    src = edge_index[0]
    dst = edge_index[1]
    h0 = jnp.concatenate(
        [x, jnp.ones((N, 1), jnp.float32), jnp.zeros((N, HID - 5), jnp.float32)],
        axis=1)

    wl0 = jnp.zeros((HID, 2 * HID), jnp.float32)
    wl0 = wl0.at[:, 0:4].set(p['init_Wl'][:, 0:4])
    wl0 = wl0.at[:, HID:HID + 4].set(p['init_Wl'][:, 4:8])
    wr0 = jnp.zeros((HID, HID), jnp.float32).at[:, 0:4].set(p['init_Wr'])
    rw0 = jnp.zeros((HID, HID), jnp.float32).at[:, 0:4].set(p['res_W'])
    eye = jnp.eye(HID, dtype=jnp.float32)
    zb = jnp.zeros((1, HID), jnp.float32)
    r2 = lambda a: a.reshape(1, -1)

    def agg(h):
        xj = jnp.take(h, src, axis=0)
        s = jax.ops.segment_sum(xj, dst, num_segments=N)
        mx = jax.ops.segment_max(xj, dst, num_segments=N)
        return s, mx

    sum0, max0 = agg(h0)
    s1 = sum0
    h = _dense(h0, sum0, max0, s1, wl0, r2(p['init_bl']), wr0, rw0,
               r2(p['res_b']), r2(p['ln_g'][0]), r2(p['ln_b'][0]))
    for i in range(N_LAYERS - 1):
        si, mi = agg(h)
        h = _dense(h, si, mi, s1, p['conv_Wl'][i], r2(p['conv_bl'][i]),
                   p['conv_Wr'][i], eye, zb,
                   r2(p['ln_g'][i + 1]), r2(p['ln_b'][i + 1]))
    m = _mlp(h, p['mlp_W1'], r2(p['mlp_b1']), r2(p['mlp_g']), r2(p['mlp_be']),
             p['mlp_W2'], r2(p['mlp_b2']))

    psum = jax.ops.segment_sum(m, batch, num_segments=G)
    pmax = jax.ops.segment_max(m, batch, num_segments=G)
    root = jnp.take(x, ptr[:G], axis=0)

    ps = jnp.pad(psum, ((0, 128 - G), (0, 0)))
    pm = jnp.pad(pmax, ((0, 128 - G), (0, 0)))
    rt = jnp.pad(root, ((0, 128 - G), (0, HID - 4)))
    ptrf = jnp.pad(ptr, (0, 128 - G - 1),
                   constant_values=N).astype(jnp.float32)

    w1 = jnp.zeros((4 * HID, 4 * HID), jnp.float32)
    w1 = w1.at[:, 0:3 * HID].set(p['ro_W1'][:, 0:3 * HID])
    w1 = w1.at[:, 3 * HID:3 * HID + 4].set(p['ro_W1'][:, 3 * HID:3 * HID + 4])
    out = _readout(ps, pm, rt, ptrf, w1, r2(p['ro_b1']), r2(p['ro_g']),
                   r2(p['ro_be']), p['ro_W2'], r2(p['ro_b2']))
    return out[:G]
